# baseline (speedup 1.0000x reference)
import functools

import jax
import jax.numpy as jnp
from jax import lax
from jax.experimental import pallas as pl
from jax.experimental.pallas import tpu as pltpu
from jax.experimental.pallas import tpu_sc as plsc

V = 50000
NF = 100000
NE = 3 * NF
NS = 16
NC = 2
EPT = 147 * 128
NEP = NS * EPT
NPAD = NEP - NE
NB = EPT // 16
LOG2M = 18
M = 1 << LOG2M
DUMP = M

_PAD = 128


def _i32(x):
    return jnp.int32(x - (1 << 32) if x >= (1 << 31) else x)

A_OWN = 0x8DA6B343
A_MUL = 0x85EBCA77
A_INIT = 0x9E3779B1


def _srl(x, n):
    return lax.shift_right_logical(x, jnp.full(x.shape, n, x.dtype))


def _make_round(ept):
    nb = ept // 16

    def body(keys3, el23, m3, amul_h,
             m3o, dsum, dcnt, avo,
             keyv, el2v, mv, slotv, scr1, stgf, stgi, amulv,
             table, keys_sh):
        c = lax.axis_index("c")
        s = lax.axis_index("s")
        wid = c * NS + s
        base = s * ept
        iota = lax.iota(jnp.int32, 16)

        pltpu.sync_copy(keys3.at[s], keyv)
        pltpu.sync_copy(el23.at[s], el2v)
        pltpu.sync_copy(m3.at[c].at[s], mv)
        pltpu.sync_copy(amul_h, amulv)
        pltpu.sync_copy(keyv, keys_sh.at[pl.ds(base, ept)])
        a_mul = amulv[pl.ds(0, 16)]

        def phase_a(i, _):
            k = keyv[pl.ds(i * 16, 16)]
            m = mv[pl.ds(i * 16, 16)]
            slot = _srl(k * a_mul, 32 - LOG2M)
            slotv[pl.ds(i * 16, 16)] = jnp.where(m == 0, slot, DUMP)
            scr1[pl.ds(i * 16, 16)] = base + i * 16 + iota
            return 0

        lax.fori_loop(0, nb, phase_a, 0)

        pltpu.sync_copy(scr1, table.at[slotv])
        plsc.subcore_barrier()
        pltpu.sync_copy(table.at[slotv], scr1)

        def phase_c(i, _):
            sl = pl.ds(i * 16, 16)
            t = scr1[sl]
            m = mv[sl]
            myidx = base + i * 16 + iota
            active = m == 0
            is_lead = jnp.logical_and(active, t == myidx)
            mv[sl] = jnp.where(is_lead, 2, m)
            slotv[sl] = jnp.where(jnp.logical_and(active, t != myidx), t, 0)
            return 0

        lax.fori_loop(0, nb, phase_c, 0)
        pltpu.sync_copy(keys_sh.at[slotv], scr1)

        def phase_d(i, carry_d):
            a_s, a_c, av = carry_d
            sl = pl.ds(i * 16, 16)
            k = keyv[sl]
            m = mv[sl]
            kt = scr1[sl]
            e = el2v[sl]
            lead = m == 2
            res = jnp.logical_or(lead, jnp.logical_and(m == 0, kt == k))
            a_s = a_s + jnp.where(lead, e, 0.0)
            a_c = a_c + jnp.where(lead, 1, 0)
            mnew = jnp.where(res, 1, m)
            mv[sl] = mnew
            av = av + jnp.where(mnew == 0, 1, 0)
            return a_s, a_c, av

        acc_s, acc_c, av = lax.fori_loop(
            0, nb, phase_d,
            (jnp.zeros((16,), jnp.float32), jnp.zeros((16,), jnp.int32),
             jnp.zeros((16,), jnp.int32)))

        pltpu.sync_copy(mv, m3o.at[c].at[s])
        stgf[0, pl.ds(0, 16)] = acc_s
        stgi[0, pl.ds(0, 16)] = acc_c
        stgi[1, pl.ds(0, 16)] = av
        pltpu.sync_copy(stgf.at[0], dsum.at[wid])
        pltpu.sync_copy(stgi.at[0], dcnt.at[wid])
        pltpu.sync_copy(stgi.at[1], avo.at[wid])

    mesh = plsc.VectorSubcoreMesh(core_axis_name="c", subcore_axis_name="s")
    return pl.kernel(
        body,
        out_type=[
            jax.ShapeDtypeStruct((NC, NS, ept), jnp.int32),
            jax.ShapeDtypeStruct((NC * NS, 16), jnp.float32),
            jax.ShapeDtypeStruct((NC * NS, 16), jnp.int32),
            jax.ShapeDtypeStruct((NC * NS, 16), jnp.int32),
        ],
        mesh=mesh,
        scratch_types=[
            pltpu.VMEM((ept,), jnp.int32),
            pltpu.VMEM((ept,), jnp.float32),
            pltpu.VMEM((ept,), jnp.int32),
            pltpu.VMEM((ept,), jnp.int32),
            pltpu.VMEM((ept,), jnp.int32),
            pltpu.VMEM((1, 16), jnp.float32),
            pltpu.VMEM((2, 16), jnp.int32),
            pltpu.VMEM((16,), jnp.int32),
            pltpu.VMEM_SHARED((M + 16,), jnp.int32),
            pltpu.VMEM_SHARED((NS * ept,), jnp.int32),
        ],
    )


TIER = 32768
EPT2 = TIER // NS


@functools.partial(jax.jit, static_argnames=())
def _dedup(keys3, el23, m3_0, keysf, el2f):
    full_call = _make_round(EPT)
    tier_call = _make_round(EPT2)

    def run_loop(call, keys3_, el23_, m3_, usum, ucnt, amul, total, stop_at):
        def cond(carry):
            return carry[4] > stop_at

        def body(carry):
            m3c, us, uc, am, _ = carry
            amul_vec = jnp.full((16,), am, jnp.int32)
            m3n, ds_, dc, av = call(keys3_, el23_, m3c, amul_vec)
            return (m3n, us + jnp.sum(ds_), uc + jnp.sum(dc),
                    am * _i32(A_MUL), jnp.sum(av))

        return lax.while_loop(cond, body, (m3_, usum, ucnt, amul, total))

    m3, usum, ucnt, amul, total = run_loop(
        full_call, keys3, el23, m3_0,
        jnp.float32(0), jnp.int32(0), _i32(A_INIT), jnp.int32(1 << 30), TIER)

    m_any = jnp.minimum(m3[0].reshape(-1), m3[1].reshape(-1))
    maskA = m_any == 0
    pos = jnp.cumsum(maskA) - 1
    tgt = jnp.where(maskA, pos, TIER).astype(jnp.int32)
    compact = (jnp.zeros((TIER + 1,), jnp.int32)
               .at[tgt].set(jnp.arange(NEP, dtype=jnp.int32))[:TIER])
    keysC = keysf[compact]
    el2C = el2f[compact]
    ownC = (keysC * _i32(A_OWN) < 0).astype(jnp.int32)
    lane = jnp.arange(TIER, dtype=jnp.int32)
    pad = lane >= total
    m3t = jnp.stack([
        jnp.where(jnp.logical_or(ownC != 0, pad), 1, 0),
        jnp.where(jnp.logical_or(ownC != 1, pad), 1, 0),
    ]).reshape(NC, NS, EPT2)

    _, usum, ucnt, _, _ = run_loop(
        tier_call, keysC.reshape(NS, EPT2), el2C.reshape(NS, EPT2), m3t,
        usum, ucnt, amul, total, 0)
    return usum, ucnt


def _final_body(usum_ref, ucnt_ref, lvx_ref, lvy_ref, lvz_ref, nw_ref,
                vx_ref, vy_ref, vz_ref, out_ref):
    edge_sum = jnp.sum(usum_ref[...])
    edge_cnt = jnp.sum(ucnt_ref[...])
    nw = nw_ref[...]
    safe = jnp.where(nw > 0, nw, 1.0)
    inv_w = jnp.where(nw > 0, 1.0 / safe, nw)
    lx = lvx_ref[...] * inv_w - vx_ref[...]
    ly = lvy_ref[...] * inv_w - vy_ref[...]
    lz = lvz_ref[...] * inv_w - vz_ref[...]
    norms = jnp.sqrt(lx * lx + ly * ly + lz * lz)
    lap_loss = jnp.sum(norms) / V
    total = 0.1 * lap_loss + 10.0 * (edge_sum / edge_cnt)
    out_ref[...] = jnp.broadcast_to(total, (1, 1))


def _pad2d(x, n):
    return jnp.zeros((n,), x.dtype).at[: x.shape[0]].set(x).reshape(n // _PAD, _PAD)


def kernel(verts, faces):
    f0, f1, f2 = faces[:, 0], faces[:, 1], faces[:, 2]
    fv = verts[faces]
    v0, v1, v2 = fv[:, 0], fv[:, 1], fv[:, 2]
    A2 = jnp.sum((v1 - v2) ** 2, axis=1)
    B2 = jnp.sum((v0 - v2) ** 2, axis=1)
    C2 = jnp.sum((v0 - v1) ** 2, axis=1)

    def ekey(a, b):
        return jnp.minimum(a, b) * 65536 + jnp.maximum(a, b)

    keys = jnp.concatenate([ekey(f0, f1), ekey(f1, f2), ekey(f2, f0)])
    el2c = jnp.concatenate([C2, A2, B2])
    pad_keys = jnp.arange(NPAD, dtype=jnp.int32) * 65536 + 65535
    keysf = jnp.concatenate([keys, pad_keys])
    el2f = jnp.concatenate([el2c, jnp.zeros((NPAD,), jnp.float32)])
    own = (keysf * _i32(A_OWN) < 0).astype(jnp.int32)
    m3_0 = jnp.stack([(own != 0).astype(jnp.int32),
                      (own != 1).astype(jnp.int32)]).reshape(NC, NS, EPT)
    usum, ucnt = _dedup(keysf.reshape(NS, EPT), el2f.reshape(NS, EPT), m3_0,
                        keysf, el2f)
    usum = usum.reshape(1, 1)
    ucntf = (ucnt - NPAD).astype(jnp.float32).reshape(1, 1)

    s2 = 0.5 * (A2 + B2 + C2)
    area = jnp.sqrt(jnp.clip(0.25 * (s2 * s2 - 0.5 * (A2 * A2 + B2 * B2 + C2 * C2)), 1e-12, None))
    cota = (B2 + C2 - A2) / area
    cotb = (A2 + C2 - B2) / area
    cotc = (A2 + B2 - C2) / area
    cot = jnp.stack([cota, cotb, cotc], axis=1) / 4.0
    ii = faces[:, jnp.array([1, 2, 0])].reshape(-1)
    jj = faces[:, jnp.array([2, 0, 1])].reshape(-1)
    w = cot.reshape(-1)
    Lv = jnp.zeros((V, 3), dtype=verts.dtype)
    Lv = Lv.at[ii].add(w[:, None] * verts[jj])
    Lv = Lv.at[jj].add(w[:, None] * verts[ii])
    norm_w = jnp.zeros((V,), dtype=verts.dtype)
    norm_w = norm_w.at[ii].add(w)
    norm_w = norm_w.at[jj].add(w)

    n_v = ((V + _PAD - 1) // _PAD) * _PAD
    args = [usum, ucntf,
            _pad2d(Lv[:, 0], n_v), _pad2d(Lv[:, 1], n_v), _pad2d(Lv[:, 2], n_v),
            _pad2d(norm_w, n_v),
            _pad2d(verts[:, 0], n_v), _pad2d(verts[:, 1], n_v), _pad2d(verts[:, 2], n_v)]
    out = pl.pallas_call(
        _final_body,
        out_shape=jax.ShapeDtypeStruct((1, 1), jnp.float32),
    )(*args)
    return out[0, 0]


# speedup vs baseline: 1.0784x; 1.0784x over previous
import functools

import jax
import jax.numpy as jnp
from jax import lax
from jax.experimental import pallas as pl
from jax.experimental.pallas import tpu as pltpu
from jax.experimental.pallas import tpu_sc as plsc

V = 50000
NF = 100000
NE = 3 * NF
NS = 16
NC = 2
EPT = 147 * 128
NEP = NS * EPT
NPAD = NEP - NE
NB = EPT // 16
LOG2M = 18
M = 1 << LOG2M
DUMP = M

_PAD = 128


def _i32(x):
    return jnp.int32(x - (1 << 32) if x >= (1 << 31) else x)

A_OWN = 0x8DA6B343
A_MUL = 0x85EBCA77
A_INIT = 0x9E3779B1


def _srl(x, n):
    return lax.shift_right_logical(x, jnp.full(x.shape, n, x.dtype))


def _make_round(ept):
    nb = ept // 16

    def body(keys3, el23, m3, amul_h,
             m3o, dsum, dcnt, avo, lists,
             keyv, el2v, mv, slotv, scr1, stgf, stgi, amulv,
             table, keys_sh):
        c = lax.axis_index("c")
        s = lax.axis_index("s")
        wid = c * NS + s
        base = s * ept
        iota = lax.iota(jnp.int32, 16)

        pltpu.sync_copy(keys3.at[s], keyv)
        pltpu.sync_copy(el23.at[s], el2v)
        pltpu.sync_copy(m3.at[c].at[s], mv)
        pltpu.sync_copy(amul_h, amulv)
        pltpu.sync_copy(keyv, keys_sh.at[pl.ds(base, ept)])
        a_mul = amulv[pl.ds(0, 16)]

        def phase_a(i, _):
            k = keyv[pl.ds(i * 16, 16)]
            m = mv[pl.ds(i * 16, 16)]
            slot = _srl(k * a_mul, 32 - LOG2M)
            slotv[pl.ds(i * 16, 16)] = jnp.where(m == 0, slot, DUMP)
            scr1[pl.ds(i * 16, 16)] = base + i * 16 + iota
            return 0

        lax.fori_loop(0, nb, phase_a, 0)

        pltpu.sync_copy(scr1, table.at[slotv])
        plsc.subcore_barrier()
        pltpu.sync_copy(table.at[slotv], scr1)

        def phase_c(i, _):
            sl = pl.ds(i * 16, 16)
            t = scr1[sl]
            m = mv[sl]
            myidx = base + i * 16 + iota
            active = m == 0
            is_lead = jnp.logical_and(active, t == myidx)
            mv[sl] = jnp.where(is_lead, 2, m)
            slotv[sl] = jnp.where(jnp.logical_and(active, t != myidx), t, 0)
            return 0

        lax.fori_loop(0, nb, phase_c, 0)
        pltpu.sync_copy(keys_sh.at[slotv], scr1)

        def phase_d(i, carry_d):
            a_s, a_c, av, off, off_b = carry_d
            sl = pl.ds(i * 16, 16)
            k = keyv[sl]
            m = mv[sl]
            kt = scr1[sl]
            e = el2v[sl]
            myidx = base + i * 16 + iota
            lead = m == 2
            res = jnp.logical_or(lead, jnp.logical_and(m == 0, kt == k))
            a_s = a_s + jnp.where(lead, e, 0.0)
            a_c = a_c + jnp.where(lead, 1, 0)
            mnew = jnp.where(res, 1, m)
            mv[sl] = mnew
            act = mnew == 0
            acti = jnp.where(act, 1, 0)
            av = av + acti
            pf = plsc.cumsum(acti) - acti
            pb = plsc.cumsum(1 - acti) - (1 - acti)
            cnt = plsc.all_reduce_population_count(act)
            ib = 16 - cnt
            pos = jnp.where(act, off + pf, ept - off_b - ib + pb)
            plsc.store_scatter(slotv, [pos], myidx)
            return a_s, a_c, av, off + cnt, off_b + ib

        acc_s, acc_c, av, _, _ = lax.fori_loop(
            0, nb, phase_d,
            (jnp.zeros((16,), jnp.float32), jnp.zeros((16,), jnp.int32),
             jnp.zeros((16,), jnp.int32), jnp.zeros((16,), jnp.int32),
             jnp.zeros((16,), jnp.int32)))

        pltpu.sync_copy(mv, m3o.at[c].at[s])
        pltpu.sync_copy(slotv, lists.at[c].at[s])
        stgf[0, pl.ds(0, 16)] = acc_s
        stgi[0, pl.ds(0, 16)] = acc_c
        stgi[1, pl.ds(0, 16)] = av
        pltpu.sync_copy(stgf.at[0], dsum.at[wid])
        pltpu.sync_copy(stgi.at[0], dcnt.at[wid])
        pltpu.sync_copy(stgi.at[1], avo.at[wid])

    mesh = plsc.VectorSubcoreMesh(core_axis_name="c", subcore_axis_name="s")
    return pl.kernel(
        body,
        out_type=[
            jax.ShapeDtypeStruct((NC, NS, ept), jnp.int32),
            jax.ShapeDtypeStruct((NC * NS, 16), jnp.float32),
            jax.ShapeDtypeStruct((NC * NS, 16), jnp.int32),
            jax.ShapeDtypeStruct((NC * NS, 16), jnp.int32),
            jax.ShapeDtypeStruct((NC, NS, ept), jnp.int32),
        ],
        mesh=mesh,
        compiler_params=pltpu.CompilerParams(needs_layout_passes=False),
        scratch_types=[
            pltpu.VMEM((ept,), jnp.int32),
            pltpu.VMEM((ept,), jnp.float32),
            pltpu.VMEM((ept,), jnp.int32),
            pltpu.VMEM((ept,), jnp.int32),
            pltpu.VMEM((ept,), jnp.int32),
            pltpu.VMEM((1, 16), jnp.float32),
            pltpu.VMEM((2, 16), jnp.int32),
            pltpu.VMEM((16,), jnp.int32),
            pltpu.VMEM_SHARED((M + 16,), jnp.int32),
            pltpu.VMEM_SHARED((NS * ept,), jnp.int32),
        ],
    )


TIER = 32768
EPT2 = TIER // NS


@functools.partial(jax.jit, static_argnames=())
def _dedup(keys3, el23, m3_0, keysf, el2f):
    full_call = _make_round(EPT)
    tier_call = _make_round(EPT2)

    def run_loop(call, ept, keys3_, el23_, m3_, usum, ucnt, amul, stop_at):
        def cond(carry):
            return jnp.sum(carry[2]) > stop_at

        def body(carry):
            m3c, _, _, us, uc, am = carry
            amul_vec = jnp.full((16,), am, jnp.int32)
            m3n, ds_, dc, av, lst = call(keys3_, el23_, m3c, amul_vec)
            return (m3n, lst, av, us + jnp.sum(ds_), uc + jnp.sum(dc),
                    am * _i32(A_MUL))

        init = (m3_, jnp.zeros((NC, NS, ept), jnp.int32),
                jnp.full((NC * NS, 16), 1 << 16, jnp.int32), usum, ucnt, amul)
        return lax.while_loop(cond, body, init)

    _, lists, av, usum, ucnt, amul = run_loop(
        full_call, EPT, keys3, el23, m3_0,
        jnp.float32(0), jnp.int32(0), _i32(A_INIT), TIER)

    counts = jnp.sum(av, axis=1)
    cum = jnp.cumsum(counts)
    offsets = cum - counts
    total = cum[-1]
    j = jnp.arange(TIER, dtype=jnp.int32)
    tile = jnp.clip(jnp.searchsorted(cum, j, side="right"), 0, NC * NS - 1)
    local = jnp.clip(j - offsets[tile], 0, EPT - 1)
    compact = lists.reshape(NC * NS, EPT)[tile, local]
    keysC = keysf[compact]
    el2C = el2f[compact]
    ownC = (keysC * _i32(A_OWN) < 0).astype(jnp.int32)
    pad = j >= total
    m3t = jnp.stack([
        jnp.where(jnp.logical_or(ownC != 0, pad), 1, 0),
        jnp.where(jnp.logical_or(ownC != 1, pad), 1, 0),
    ]).reshape(NC, NS, EPT2)

    _, _, _, usum, ucnt, _ = run_loop(
        tier_call, EPT2, keysC.reshape(NS, EPT2), el2C.reshape(NS, EPT2), m3t,
        usum, ucnt, amul, 0)
    return usum, ucnt


def _final_body(usum_ref, ucnt_ref, lvx_ref, lvy_ref, lvz_ref, nw_ref,
                vx_ref, vy_ref, vz_ref, out_ref):
    edge_sum = jnp.sum(usum_ref[...])
    edge_cnt = jnp.sum(ucnt_ref[...])
    nw = nw_ref[...]
    safe = jnp.where(nw > 0, nw, 1.0)
    inv_w = jnp.where(nw > 0, 1.0 / safe, nw)
    lx = lvx_ref[...] * inv_w - vx_ref[...]
    ly = lvy_ref[...] * inv_w - vy_ref[...]
    lz = lvz_ref[...] * inv_w - vz_ref[...]
    norms = jnp.sqrt(lx * lx + ly * ly + lz * lz)
    lap_loss = jnp.sum(norms) / V
    total = 0.1 * lap_loss + 10.0 * (edge_sum / edge_cnt)
    out_ref[...] = jnp.broadcast_to(total, (1, 1))


def _pad2d(x, n):
    return jnp.zeros((n,), x.dtype).at[: x.shape[0]].set(x).reshape(n // _PAD, _PAD)


def kernel(verts, faces):
    f0, f1, f2 = faces[:, 0], faces[:, 1], faces[:, 2]
    fv = verts[faces]
    v0, v1, v2 = fv[:, 0], fv[:, 1], fv[:, 2]
    A2 = jnp.sum((v1 - v2) ** 2, axis=1)
    B2 = jnp.sum((v0 - v2) ** 2, axis=1)
    C2 = jnp.sum((v0 - v1) ** 2, axis=1)

    def ekey(a, b):
        return jnp.minimum(a, b) * 65536 + jnp.maximum(a, b)

    keys = jnp.concatenate([ekey(f0, f1), ekey(f1, f2), ekey(f2, f0)])
    el2c = jnp.concatenate([C2, A2, B2])
    pad_keys = jnp.arange(NPAD, dtype=jnp.int32) * 65536 + 65535
    keysf = jnp.concatenate([keys, pad_keys])
    el2f = jnp.concatenate([el2c, jnp.zeros((NPAD,), jnp.float32)])
    own = (keysf * _i32(A_OWN) < 0).astype(jnp.int32)
    m3_0 = jnp.stack([(own != 0).astype(jnp.int32),
                      (own != 1).astype(jnp.int32)]).reshape(NC, NS, EPT)
    usum, ucnt = _dedup(keysf.reshape(NS, EPT), el2f.reshape(NS, EPT), m3_0,
                        keysf, el2f)
    usum = usum.reshape(1, 1)
    ucntf = (ucnt - NPAD).astype(jnp.float32).reshape(1, 1)

    s2 = 0.5 * (A2 + B2 + C2)
    area = jnp.sqrt(jnp.clip(0.25 * (s2 * s2 - 0.5 * (A2 * A2 + B2 * B2 + C2 * C2)), 1e-12, None))
    cota = (B2 + C2 - A2) / area
    cotb = (A2 + C2 - B2) / area
    cotc = (A2 + B2 - C2) / area
    cot = jnp.stack([cota, cotb, cotc], axis=1) / 4.0
    ii = faces[:, jnp.array([1, 2, 0])].reshape(-1)
    jj = faces[:, jnp.array([2, 0, 1])].reshape(-1)
    w = cot.reshape(-1)
    Lv = jnp.zeros((V, 3), dtype=verts.dtype)
    Lv = Lv.at[ii].add(w[:, None] * verts[jj])
    Lv = Lv.at[jj].add(w[:, None] * verts[ii])
    norm_w = jnp.zeros((V,), dtype=verts.dtype)
    norm_w = norm_w.at[ii].add(w)
    norm_w = norm_w.at[jj].add(w)

    n_v = ((V + _PAD - 1) // _PAD) * _PAD
    args = [usum, ucntf,
            _pad2d(Lv[:, 0], n_v), _pad2d(Lv[:, 1], n_v), _pad2d(Lv[:, 2], n_v),
            _pad2d(norm_w, n_v),
            _pad2d(verts[:, 0], n_v), _pad2d(verts[:, 1], n_v), _pad2d(verts[:, 2], n_v)]
    out = pl.pallas_call(
        _final_body,
        out_shape=jax.ShapeDtypeStruct((1, 1), jnp.float32),
    )(*args)
    return out[0, 0]


# speedup vs baseline: 1.3578x; 1.2592x over previous
import functools

import jax
import jax.numpy as jnp
from jax import lax
from jax.experimental import pallas as pl
from jax.experimental.pallas import tpu as pltpu
from jax.experimental.pallas import tpu_sc as plsc

V = 50000
NF = 100000
NE = 3 * NF
NS = 16
NC = 2
EPT = 147 * 128
NEP = NS * EPT
NPAD = NEP - NE
NB = EPT // 16
LOG2M = 18
M = 1 << LOG2M
DUMP = M

_PAD = 128


def _i32(x):
    return jnp.int32(x - (1 << 32) if x >= (1 << 31) else x)

A_OWN = 0x8DA6B343
A_MUL = 0x85EBCA77
A_INIT = 0x9E3779B1


def _srl(x, n):
    return lax.shift_right_logical(x, jnp.full(x.shape, n, x.dtype))


def _make_round(ept):
    nb = ept // 16

    def body(keys3, el23, m3, amul_h,
             m3o, dsum, dcnt, avo, lists,
             keyv, el2v, mv, slotv, scr1, stgf, stgi, amulv,
             table, keys_sh):
        c = lax.axis_index("c")
        s = lax.axis_index("s")
        wid = c * NS + s
        base = s * ept
        iota = lax.iota(jnp.int32, 16)

        pltpu.sync_copy(keys3.at[s], keyv)
        pltpu.sync_copy(el23.at[s], el2v)
        pltpu.sync_copy(m3.at[c].at[s], mv)
        pltpu.sync_copy(amul_h, amulv)
        pltpu.sync_copy(keyv, keys_sh.at[pl.ds(base, ept)])
        a_mul = amulv[pl.ds(0, 16)]

        def phase_a(i, _):
            k = keyv[pl.ds(i * 16, 16)]
            m = mv[pl.ds(i * 16, 16)]
            slot = _srl(k * a_mul, 32 - LOG2M)
            slotv[pl.ds(i * 16, 16)] = jnp.where(m == 0, slot, DUMP)
            scr1[pl.ds(i * 16, 16)] = base + i * 16 + iota
            return 0

        lax.fori_loop(0, nb, phase_a, 0)

        pltpu.sync_copy(scr1, table.at[slotv])
        plsc.subcore_barrier()
        pltpu.sync_copy(table.at[slotv], scr1)

        def phase_c(i, _):
            sl = pl.ds(i * 16, 16)
            t = scr1[sl]
            m = mv[sl]
            myidx = base + i * 16 + iota
            active = m == 0
            is_lead = jnp.logical_and(active, t == myidx)
            mv[sl] = jnp.where(is_lead, 2, m)
            slotv[sl] = jnp.where(jnp.logical_and(active, t != myidx), t, 0)
            return 0

        lax.fori_loop(0, nb, phase_c, 0)
        pltpu.sync_copy(keys_sh.at[slotv], scr1)

        def phase_d(i, carry_d):
            a_s, a_c, av, off, off_b = carry_d
            sl = pl.ds(i * 16, 16)
            k = keyv[sl]
            m = mv[sl]
            kt = scr1[sl]
            e = el2v[sl]
            myidx = base + i * 16 + iota
            lead = m == 2
            res = jnp.logical_or(lead, jnp.logical_and(m == 0, kt == k))
            a_s = a_s + jnp.where(lead, e, 0.0)
            a_c = a_c + jnp.where(lead, 1, 0)
            mnew = jnp.where(res, 1, m)
            mv[sl] = mnew
            act = mnew == 0
            acti = jnp.where(act, 1, 0)
            av = av + acti
            pf = plsc.cumsum(acti) - acti
            pb = plsc.cumsum(1 - acti) - (1 - acti)
            cnt = plsc.all_reduce_population_count(act)
            ib = 16 - cnt
            pos = jnp.where(act, off + pf, ept - off_b - ib + pb)
            plsc.store_scatter(slotv, [pos], myidx)
            return a_s, a_c, av, off + cnt, off_b + ib

        acc_s, acc_c, av, _, _ = lax.fori_loop(
            0, nb, phase_d,
            (jnp.zeros((16,), jnp.float32), jnp.zeros((16,), jnp.int32),
             jnp.zeros((16,), jnp.int32), jnp.zeros((16,), jnp.int32),
             jnp.zeros((16,), jnp.int32)))

        pltpu.sync_copy(mv, m3o.at[c].at[s])
        pltpu.sync_copy(slotv, lists.at[c].at[s])
        stgf[0, pl.ds(0, 16)] = acc_s
        stgi[0, pl.ds(0, 16)] = acc_c
        stgi[1, pl.ds(0, 16)] = av
        pltpu.sync_copy(stgf.at[0], dsum.at[wid])
        pltpu.sync_copy(stgi.at[0], dcnt.at[wid])
        pltpu.sync_copy(stgi.at[1], avo.at[wid])

    mesh = plsc.VectorSubcoreMesh(core_axis_name="c", subcore_axis_name="s")
    return pl.kernel(
        body,
        out_type=[
            jax.ShapeDtypeStruct((NC, NS, ept), jnp.int32),
            jax.ShapeDtypeStruct((NC * NS, 16), jnp.float32),
            jax.ShapeDtypeStruct((NC * NS, 16), jnp.int32),
            jax.ShapeDtypeStruct((NC * NS, 16), jnp.int32),
            jax.ShapeDtypeStruct((NC, NS, ept), jnp.int32),
        ],
        mesh=mesh,
        compiler_params=pltpu.CompilerParams(needs_layout_passes=False),
        scratch_types=[
            pltpu.VMEM((ept,), jnp.int32),
            pltpu.VMEM((ept,), jnp.float32),
            pltpu.VMEM((ept,), jnp.int32),
            pltpu.VMEM((ept,), jnp.int32),
            pltpu.VMEM((ept,), jnp.int32),
            pltpu.VMEM((1, 16), jnp.float32),
            pltpu.VMEM((2, 16), jnp.int32),
            pltpu.VMEM((16,), jnp.int32),
            pltpu.VMEM_SHARED((M + 16,), jnp.int32),
            pltpu.VMEM_SHARED((NS * ept,), jnp.int32),
        ],
    )


TIER = 32768
EPT2 = TIER // NS


@functools.partial(jax.jit, static_argnames=())
def _dedup(keys3, el23, m3_0, keysf, el2f):
    full_call = _make_round(EPT)
    tier_call = _make_round(EPT2)

    def run_loop(call, ept, keys3_, el23_, m3_, usum, ucnt, amul, stop_at):
        def cond(carry):
            return jnp.sum(carry[2]) > stop_at

        def body(carry):
            m3c, _, _, us, uc, am = carry
            amul_vec = jnp.full((16,), am, jnp.int32)
            m3n, ds_, dc, av, lst = call(keys3_, el23_, m3c, amul_vec)
            return (m3n, lst, av, us + jnp.sum(ds_), uc + jnp.sum(dc),
                    am * _i32(A_MUL))

        init = (m3_, jnp.zeros((NC, NS, ept), jnp.int32),
                jnp.full((NC * NS, 16), 1 << 16, jnp.int32), usum, ucnt, amul)
        return lax.while_loop(cond, body, init)

    _, lists, av, usum, ucnt, amul = run_loop(
        full_call, EPT, keys3, el23, m3_0,
        jnp.float32(0), jnp.int32(0), _i32(A_INIT), TIER)

    counts = jnp.sum(av, axis=1)
    cum = jnp.cumsum(counts)
    offsets = cum - counts
    total = cum[-1]
    j = jnp.arange(TIER, dtype=jnp.int32)
    tile = jnp.clip(jnp.searchsorted(cum, j, side="right"), 0, NC * NS - 1)
    local = jnp.clip(j - offsets[tile], 0, EPT - 1)
    compact = lists.reshape(NC * NS, EPT)[tile, local]
    keysC = keysf[compact]
    el2C = el2f[compact]
    ownC = (keysC * _i32(A_OWN) < 0).astype(jnp.int32)
    pad = j >= total
    m3t = jnp.stack([
        jnp.where(jnp.logical_or(ownC != 0, pad), 1, 0),
        jnp.where(jnp.logical_or(ownC != 1, pad), 1, 0),
    ]).reshape(NC, NS, EPT2)

    _, _, _, usum, ucnt, _ = run_loop(
        tier_call, EPT2, keysC.reshape(NS, EPT2), el2C.reshape(NS, EPT2), m3t,
        usum, ucnt, amul, 0)
    return usum, ucnt


VP = 51200
ET2 = 18752
NEP2 = 32 * ET2
STRIPE = VP // NS


def _scatter_body(tgt2, vx2, vy2, vz2, vw2, zacc, out,
                  tgtv, vxv, vyv, vzv, vwv, acc0, acc1, acc2, acc3):
    c = lax.axis_index("c")
    s = lax.axis_index("s")
    wid = c * NS + s
    accs = (acc0, acc1, acc2, acc3)
    pltpu.sync_copy(tgt2.at[wid], tgtv)
    pltpu.sync_copy(vx2.at[wid], vxv)
    pltpu.sync_copy(vy2.at[wid], vyv)
    pltpu.sync_copy(vz2.at[wid], vzv)
    pltpu.sync_copy(vw2.at[wid], vwv)
    for k in range(4):
        pltpu.sync_copy(zacc.at[k].at[pl.ds(s * STRIPE, STRIPE)],
                        accs[k].at[pl.ds(s * STRIPE, STRIPE)])
    plsc.subcore_barrier()
    pltpu.sync_copy(vxv, acc0.at[tgtv], add=True)
    pltpu.sync_copy(vyv, acc1.at[tgtv], add=True)
    pltpu.sync_copy(vzv, acc2.at[tgtv], add=True)
    pltpu.sync_copy(vwv, acc3.at[tgtv], add=True)
    plsc.subcore_barrier()
    for k in range(4):
        pltpu.sync_copy(accs[k].at[pl.ds(s * STRIPE, STRIPE)],
                        out.at[c].at[k].at[pl.ds(s * STRIPE, STRIPE)])


@jax.jit
def _scatter_add(tgt2, vx2, vy2, vz2, vw2, zacc):
    mesh = plsc.VectorSubcoreMesh(core_axis_name="c", subcore_axis_name="s")
    return pl.kernel(
        _scatter_body,
        out_type=jax.ShapeDtypeStruct((NC, 4, VP), jnp.float32),
        mesh=mesh,
        compiler_params=pltpu.CompilerParams(needs_layout_passes=False),
        scratch_types=[
            pltpu.VMEM((ET2,), jnp.int32),
            pltpu.VMEM((ET2,), jnp.float32),
            pltpu.VMEM((ET2,), jnp.float32),
            pltpu.VMEM((ET2,), jnp.float32),
            pltpu.VMEM((ET2,), jnp.float32),
            pltpu.VMEM_SHARED((VP,), jnp.float32),
            pltpu.VMEM_SHARED((VP,), jnp.float32),
            pltpu.VMEM_SHARED((VP,), jnp.float32),
            pltpu.VMEM_SHARED((VP,), jnp.float32),
        ],
    )(tgt2, vx2, vy2, vz2, vw2, zacc)


def _final_body(usum_ref, ucnt_ref, lvx_ref, lvy_ref, lvz_ref, nw_ref,
                vx_ref, vy_ref, vz_ref, out_ref):
    edge_sum = jnp.sum(usum_ref[...])
    edge_cnt = jnp.sum(ucnt_ref[...])
    nw = nw_ref[...]
    shp = nw.shape
    idx = (lax.broadcasted_iota(jnp.int32, shp, 0) * shp[1]
           + lax.broadcasted_iota(jnp.int32, shp, 1))
    valid = idx < V
    safe = jnp.where(nw > 0, nw, 1.0)
    inv_w = jnp.where(nw > 0, 1.0 / safe, nw)
    lx = lvx_ref[...] * inv_w - vx_ref[...]
    ly = lvy_ref[...] * inv_w - vy_ref[...]
    lz = lvz_ref[...] * inv_w - vz_ref[...]
    norms = jnp.where(valid, jnp.sqrt(lx * lx + ly * ly + lz * lz), 0.0)
    lap_loss = jnp.sum(norms) / V
    total = 0.1 * lap_loss + 10.0 * (edge_sum / edge_cnt)
    out_ref[...] = jnp.broadcast_to(total, (1, 1))


def _pad2d(x, n):
    return jnp.zeros((n,), x.dtype).at[: x.shape[0]].set(x).reshape(n // _PAD, _PAD)


def kernel(verts, faces):
    f0, f1, f2 = faces[:, 0], faces[:, 1], faces[:, 2]
    fv = verts[faces]
    v0, v1, v2 = fv[:, 0], fv[:, 1], fv[:, 2]
    A2 = jnp.sum((v1 - v2) ** 2, axis=1)
    B2 = jnp.sum((v0 - v2) ** 2, axis=1)
    C2 = jnp.sum((v0 - v1) ** 2, axis=1)

    def ekey(a, b):
        return jnp.minimum(a, b) * 65536 + jnp.maximum(a, b)

    keys = jnp.concatenate([ekey(f0, f1), ekey(f1, f2), ekey(f2, f0)])
    el2c = jnp.concatenate([C2, A2, B2])
    pad_keys = jnp.arange(NPAD, dtype=jnp.int32) * 65536 + 65535
    keysf = jnp.concatenate([keys, pad_keys])
    el2f = jnp.concatenate([el2c, jnp.zeros((NPAD,), jnp.float32)])
    own = (keysf * _i32(A_OWN) < 0).astype(jnp.int32)
    m3_0 = jnp.stack([(own != 0).astype(jnp.int32),
                      (own != 1).astype(jnp.int32)]).reshape(NC, NS, EPT)
    usum, ucnt = _dedup(keysf.reshape(NS, EPT), el2f.reshape(NS, EPT), m3_0,
                        keysf, el2f)
    usum = usum.reshape(1, 1)
    ucntf = (ucnt - NPAD).astype(jnp.float32).reshape(1, 1)

    s2 = 0.5 * (A2 + B2 + C2)
    area = jnp.sqrt(jnp.clip(0.25 * (s2 * s2 - 0.5 * (A2 * A2 + B2 * B2 + C2 * C2)), 1e-12, None))
    cota = (B2 + C2 - A2) / area
    cotb = (A2 + C2 - B2) / area
    cotc = (A2 + B2 - C2) / area
    cot = jnp.stack([cota, cotb, cotc], axis=1) / 4.0
    ii = faces[:, jnp.array([1, 2, 0])].reshape(-1)
    jj = faces[:, jnp.array([2, 0, 1])].reshape(-1)
    w = cot.reshape(-1)
    tgt = jnp.concatenate([ii, jj, jnp.full((64,), V, jnp.int32)])
    z64 = jnp.zeros((64,), jnp.float32)
    wvj = w[:, None] * verts[jj]
    wvi = w[:, None] * verts[ii]
    vx = jnp.concatenate([wvj[:, 0], wvi[:, 0], z64]).reshape(NC * NS, ET2)
    vy = jnp.concatenate([wvj[:, 1], wvi[:, 1], z64]).reshape(NC * NS, ET2)
    vz = jnp.concatenate([wvj[:, 2], wvi[:, 2], z64]).reshape(NC * NS, ET2)
    vw = jnp.concatenate([w, w, z64]).reshape(NC * NS, ET2)
    acc2 = _scatter_add(tgt.reshape(NC * NS, ET2), vx, vy, vz, vw,
                        jnp.zeros((4, VP), jnp.float32))
    acc = acc2[0] + acc2[1]

    n_v = VP
    args = [usum, ucntf,
            acc[0].reshape(n_v // _PAD, _PAD),
            acc[1].reshape(n_v // _PAD, _PAD),
            acc[2].reshape(n_v // _PAD, _PAD),
            acc[3].reshape(n_v // _PAD, _PAD),
            _pad2d(verts[:, 0], n_v), _pad2d(verts[:, 1], n_v), _pad2d(verts[:, 2], n_v)]
    out = pl.pallas_call(
        _final_body,
        out_shape=jax.ShapeDtypeStruct((1, 1), jnp.float32),
    )(*args)
    return out[0, 0]
